# baseline (speedup 1.0000x reference)
import jax
import jax.numpy as jnp
from jax import lax
from jax.experimental import pallas as pl
from jax.experimental.pallas import tpu as pltpu
from jax.experimental.pallas import tpu_sc as plsc

NUM_ITEMS = 1000000
EMBED = 32
CTX = 32
B = 4096
L_ITEM = 20
L_CTX = 200

NC = 2
NS = 16
NW = NC * NS
BPW = B // NW

SUP = NUM_ITEMS // 4
CB = 2
CROWS = CB * L_CTX
NCH = BPW // CB
CTX_PER_W = BPW * L_CTX

IROWS_PER_W = BPW * L_ITEM
ICHUNK = 256
NICH = IROWS_PER_W // ICHUNK

_SELU_ALPHA = 1.6732632423543772
_SELU_SCALE = 1.0507009873554805


def _sc_ctx_body(sup_hbm, addr_hbm, emb_hbm, summed_hbm,
                 sup0, sup1, adr0, adr1, rows0, rows1, acc_v, sem0, sem1):
    c = lax.axis_index("c")
    s = lax.axis_index("s")
    w = s * NC + c
    base = w * CTX_PER_W

    def load_idx(g, sup_v, adr_v):
        off = pl.multiple_of(base + g * CROWS, 8)
        pltpu.sync_copy(sup_hbm.at[pl.ds(off, CROWS)], sup_v)
        pltpu.sync_copy(addr_hbm.at[pl.ds(off, CROWS)],
                        adr_v.at[pl.ds(0, CROWS)])

    def start_gather(sup_v, rows_v, sem):
        pltpu.async_copy(emb_hbm.at[sup_v], rows_v, sem)

    def wait_gather(rows_v, sem):
        pltpu.make_async_copy(emb_hbm.at[pl.ds(0, CROWS)], rows_v, sem).wait()

    def accumulate(g, adr_v, rows_v):
        z = jnp.zeros((16,), jnp.float32)
        for r in range(CB):
            def body(j, carry, r=r):
                a0, a1 = carry
                av = adr_v[pl.ds(r * L_CTX + j, 16)]
                for u in range(8):
                    aw = av[u]
                    row = lax.shift_right_logical(aw, 7)
                    col = lax.bitwise_and(aw, 127)
                    a0 = a0 + rows_v[row, pl.ds(col, 16)]
                    a1 = a1 + rows_v[row, pl.ds(col + 16, 16)]
                return (a0, a1)

            a0, a1 = plsc.parallel_loop(0, L_CTX, step=8, unroll=1,
                                        carry=(z, z))(body)
            acc_v[g * CB + r, pl.ds(0, 16)] = a0
            acc_v[g * CB + r, pl.ds(16, 16)] = a1

    load_idx(0, sup0, adr0)
    start_gather(sup0, rows0, sem0)

    def loop(t, _):
        ga = 2 * t
        gb = 2 * t + 1
        load_idx(gb, sup1, adr1)
        start_gather(sup1, rows1, sem1)
        wait_gather(rows0, sem0)
        accumulate(ga, adr0, rows0)

        @pl.when(t < NCH // 2 - 1)
        def _():
            load_idx(ga + 2, sup0, adr0)
            start_gather(sup0, rows0, sem0)

        wait_gather(rows1, sem1)
        accumulate(gb, adr1, rows1)
        return 0

    lax.fori_loop(0, NCH // 2, loop, 0)
    pltpu.sync_copy(acc_v, summed_hbm.at[pl.ds(pl.multiple_of(w * BPW, 8), BPW)])


def _sc_ctx(ctx_sup, ctx_addr, emb_lin):
    mesh = plsc.VectorSubcoreMesh(core_axis_name="c", subcore_axis_name="s")
    return pl.kernel(
        _sc_ctx_body,
        out_type=jax.ShapeDtypeStruct((B, EMBED), jnp.float32),
        mesh=mesh,
        scratch_types=[
            pltpu.VMEM((CROWS,), jnp.int32),
            pltpu.VMEM((CROWS,), jnp.int32),
            pltpu.VMEM((CROWS + 8,), jnp.int32),
            pltpu.VMEM((CROWS + 8,), jnp.int32),
            pltpu.VMEM((CROWS, 128), jnp.float32),
            pltpu.VMEM((CROWS, 128), jnp.float32),
            pltpu.VMEM((BPW, EMBED), jnp.float32),
            pltpu.SemaphoreType.DMA,
            pltpu.SemaphoreType.DMA,
        ],
    )(ctx_sup, ctx_addr, emb_lin)


def _sc_item_body(sup_hbm, itm_hbm, out_hbm, iidx_v, rows0, rows1, sem0, sem1):
    c = lax.axis_index("c")
    s = lax.axis_index("s")
    w = s * NC + c
    base = w * IROWS_PER_W
    pltpu.sync_copy(sup_hbm.at[pl.ds(pl.multiple_of(base, 8), IROWS_PER_W)],
                    iidx_v)

    def start(ch, rows_v, sem):
        pltpu.async_copy(itm_hbm.at[iidx_v.at[pl.ds(ch * ICHUNK, ICHUNK)]],
                         rows_v, sem)

    def drain(ch, rows_v, sem):
        pltpu.make_async_copy(itm_hbm.at[pl.ds(0, ICHUNK)], rows_v, sem).wait()
        off = pl.multiple_of(base + ch * ICHUNK, 8)
        pltpu.sync_copy(rows_v, out_hbm.at[pl.ds(off, ICHUNK)])

    start(0, rows0, sem0)

    def loop(t, _):
        start(2 * t + 1, rows1, sem1)
        drain(2 * t, rows0, sem0)

        @pl.when(t < NICH // 2 - 1)
        def _():
            start(2 * t + 2, rows0, sem0)

        drain(2 * t + 1, rows1, sem1)
        return 0

    lax.fori_loop(0, NICH // 2, loop, 0)


def _sc_item(item_sup, itm_lin):
    mesh = plsc.VectorSubcoreMesh(core_axis_name="c", subcore_axis_name="s")
    return pl.kernel(
        _sc_item_body,
        out_type=jax.ShapeDtypeStruct((B * L_ITEM, 128), jnp.float32),
        mesh=mesh,
        scratch_types=[
            pltpu.VMEM((IROWS_PER_W,), jnp.int32),
            pltpu.VMEM((ICHUNK, 128), jnp.float32),
            pltpu.VMEM((ICHUNK, 128), jnp.float32),
            pltpu.SemaphoreType.DMA,
            pltpu.SemaphoreType.DMA,
        ],
    )(item_sup, itm_lin)


def _selu(x):
    return _SELU_SCALE * jnp.where(x > 0, x, _SELU_ALPHA * (jnp.exp(x) - 1.0))


def _tc_body(summed_ref, iidx_ref, irows_ref,
             W1_ref, b1_ref, W2_ref, b2_ref, W3_ref, b3_ref, out_ref):
    s = summed_ref[...]
    sq = jnp.sum(s * s, axis=-1, keepdims=True)
    n = s * lax.rsqrt(jnp.maximum(sq, 1e-4))
    h = _selu(jnp.dot(n, W1_ref[...], preferred_element_type=jnp.float32)
              + b1_ref[...])
    h = _selu(jnp.dot(h, W2_ref[...], preferred_element_type=jnp.float32)
              + b2_ref[...])
    ce = (jnp.dot(h, W3_ref[...], preferred_element_type=jnp.float32)
          + b3_ref[...])

    irows = irows_ref[...]
    slots = jnp.bitwise_and(iidx_ref[...], 3)
    rep_r = lax.broadcasted_iota(jnp.int32, (L_ITEM, L_ITEM * EMBED), 0)
    rep_c = lax.broadcasted_iota(jnp.int32, (L_ITEM, L_ITEM * EMBED), 1)
    R = jnp.where(rep_c // EMBED == rep_r, 1.0, 0.0).astype(jnp.float32)
    items32 = jnp.zeros_like(ce, shape=(ce.shape[0], L_ITEM * EMBED))
    for k in range(4):
        mk = jnp.where(slots == k, 1.0, 0.0).astype(jnp.float32)
        mexp = jnp.dot(mk, R, preferred_element_type=jnp.float32)
        sel = jnp.concatenate(
            [irows[:, i * 128 + k * EMBED: i * 128 + (k + 1) * EMBED]
             for i in range(L_ITEM)], axis=1)
        items32 = items32 + mexp * sel

    cet = jnp.concatenate([ce] * L_ITEM, axis=1)
    dvec = items32 - cet
    d2 = dvec * dvec
    g_r = lax.broadcasted_iota(jnp.int32, (L_ITEM * EMBED, L_ITEM), 0)
    g_c = lax.broadcasted_iota(jnp.int32, (L_ITEM * EMBED, L_ITEM), 1)
    G = jnp.where(g_r // EMBED == g_c, 1.0, 0.0).astype(jnp.float32)
    d = jnp.dot(d2, G, preferred_element_type=jnp.float32)
    out_ref[...] = 1.0 / (1.0 + d)


def _tc_tail(summed, item_idx, irows, W1, b1, W2, b2, W3, b3):
    BT = 256
    grid = (B // BT,)
    full = lambda shape: pl.BlockSpec(shape, lambda i: (0, 0))
    return pl.pallas_call(
        _tc_body,
        grid=grid,
        in_specs=[
            pl.BlockSpec((BT, EMBED), lambda i: (i, 0)),
            pl.BlockSpec((BT, L_ITEM), lambda i: (i, 0)),
            pl.BlockSpec((BT, L_ITEM * 128), lambda i: (i, 0)),
            full((CTX, 2 * CTX)),
            full((1, 2 * CTX)),
            full((2 * CTX, 4 * CTX)),
            full((1, 4 * CTX)),
            full((4 * CTX, EMBED)),
            full((1, EMBED)),
        ],
        out_specs=pl.BlockSpec((BT, L_ITEM), lambda i: (i, 0)),
        out_shape=jax.ShapeDtypeStruct((B, L_ITEM), jnp.float32),
    )(summed, item_idx, irows, W1, b1, W2, b2, W3, b3)


def kernel(item_indices, context_indices, item_table, set_table,
           W1, b1, W2, b2, W3, b3):
    ctx_i32 = context_indices.astype(jnp.int32)
    itm_i32 = item_indices.astype(jnp.int32)

    emb_lin = jnp.concatenate(
        [jnp.zeros((1, EMBED), jnp.float32), set_table], axis=0
    ).reshape(SUP, 128)
    itm_lin = item_table.reshape(SUP, 128)

    ctx_flat = ctx_i32.reshape(-1)
    ctx_sup = lax.shift_right_logical(ctx_flat, 2)
    chunk_pos = jnp.arange(B * L_CTX, dtype=jnp.int32) % CROWS
    ctx_addr = chunk_pos * 128 + jnp.bitwise_and(ctx_flat, 3) * EMBED

    item_flat = itm_i32.reshape(-1)
    item_sup = lax.shift_right_logical(item_flat, 2)

    summed = _sc_ctx(ctx_sup, ctx_addr, emb_lin)
    irows = _sc_item(item_sup, itm_lin)
    return _tc_tail(summed, itm_i32, irows.reshape(B, L_ITEM * 128),
                    W1, b1.reshape(1, -1), W2, b2.reshape(1, -1),
                    W3, b3.reshape(1, -1))


# speedup vs baseline: 1.3753x; 1.3753x over previous
import functools

import jax
import jax.numpy as jnp
from jax import lax
from jax.experimental import pallas as pl
from jax.experimental.pallas import tpu as pltpu
from jax.experimental.pallas import tpu_sc as plsc

NUM_ITEMS = 1000000
EMBED = 32
CTX = 32
B = 4096
L_ITEM = 20
L_CTX = 200

NC = 2
NS = 16
NW = NC * NS
BPW = B // NW
CB = 8
N_CCHUNK = BPW // CB
CROWS = CB * L_CTX
IROWS_PER_W = BPW * L_ITEM
ICHUNK = 512
N_ICHUNK = IROWS_PER_W // ICHUNK

_SELU_ALPHA = 1.6732632423543772
_SELU_SCALE = 1.0507009873554805


def _sc_body(item_idx_hbm, ctx_idx_hbm, item_tab_hbm, set_tab_hbm,
             summed_hbm, irows_out_hbm,
             cidx_v, crows_v, acc_v, iidx_v, irows_v, sem):
    c = lax.axis_index("c")
    s = lax.axis_index("s")
    w = s * NC + c
    cbase = w * (BPW * L_CTX)
    ibase = w * IROWS_PER_W

    pltpu.sync_copy(item_idx_hbm.at[pl.ds(pl.multiple_of(ibase, 8), IROWS_PER_W)],
                    iidx_v)

    def item_chunk(k, _):
        off = pl.multiple_of(ibase + k * ICHUNK, 8)
        pltpu.async_copy(item_tab_hbm.at[iidx_v.at[pl.ds(k * ICHUNK, ICHUNK)]],
                         irows_v, sem).wait()
        pltpu.sync_copy(irows_v, irows_out_hbm.at[pl.ds(off, ICHUNK)])
        return 0

    lax.fori_loop(0, N_ICHUNK, item_chunk, 0)

    def ctx_chunk(g, _):
        goff = pl.multiple_of(cbase + g * CROWS, 8)
        pltpu.sync_copy(ctx_idx_hbm.at[pl.ds(goff, CROWS)], cidx_v)
        pltpu.async_copy(set_tab_hbm.at[cidx_v], crows_v, sem).wait()
        for r in range(CB):
            z = jnp.zeros((16,), jnp.float32)

            def body(j, carry, r=r):
                a0, a1 = carry
                row = r * L_CTX + j
                a0 = a0 + crows_v[row, pl.ds(0, 16)]
                a1 = a1 + crows_v[row, pl.ds(16, 16)]
                return (a0, a1)

            a0, a1 = plsc.parallel_loop(0, L_CTX, unroll=8, carry=(z, z))(body)
            acc_v[g * CB + r, pl.ds(0, 16)] = a0
            acc_v[g * CB + r, pl.ds(16, 16)] = a1
        return 0

    lax.fori_loop(0, N_CCHUNK, ctx_chunk, 0)

    pltpu.sync_copy(acc_v, summed_hbm.at[pl.ds(pl.multiple_of(w * BPW, 8), BPW)])


def _sc_gather_pool(item_idx, ctx_idx, item_table, set_table):
    mesh = plsc.VectorSubcoreMesh(core_axis_name="c", subcore_axis_name="s")
    return pl.kernel(
        _sc_body,
        out_type=[
            jax.ShapeDtypeStruct((B, EMBED), jnp.float32),
            jax.ShapeDtypeStruct((B * L_ITEM, EMBED), jnp.float32),
        ],
        mesh=mesh,
        scratch_types=[
            pltpu.VMEM((CROWS,), jnp.int32),
            pltpu.VMEM((CROWS, EMBED), jnp.float32),
            pltpu.VMEM((BPW, EMBED), jnp.float32),
            pltpu.VMEM((IROWS_PER_W,), jnp.int32),
            pltpu.VMEM((ICHUNK, EMBED), jnp.float32),
            pltpu.SemaphoreType.DMA,
        ],
        compiler_params=pltpu.CompilerParams(use_tc_tiling_on_sc=False),
    )(item_idx, ctx_idx, item_table, set_table)


def _selu(x):
    return _SELU_SCALE * jnp.where(x > 0, x, _SELU_ALPHA * (jnp.exp(x) - 1.0))


def _tc_body(summed_ref, ctx_idx_ref, st0_ref, irows_ref,
             W1_ref, b1_ref, W2_ref, b2_ref, W3_ref, b3_ref, out_ref):
    summed = summed_ref[...]
    idx = ctx_idx_ref[...]
    zcnt = jnp.sum(jnp.where(idx == 0, 1.0, 0.0).astype(jnp.float32),
                   axis=1, keepdims=True)
    s = summed - zcnt * st0_ref[...]
    sq = jnp.sum(s * s, axis=-1, keepdims=True)
    n = s * lax.rsqrt(jnp.maximum(sq, 1e-4))
    h = _selu(jnp.dot(n, W1_ref[...], preferred_element_type=jnp.float32)
              + b1_ref[...])
    h = _selu(jnp.dot(h, W2_ref[...], preferred_element_type=jnp.float32)
              + b2_ref[...])
    ce = (jnp.dot(h, W3_ref[...], preferred_element_type=jnp.float32)
          + b3_ref[...])
    items = irows_ref[...]
    cet = jnp.concatenate([ce] * L_ITEM, axis=1)
    d2 = (items - cet) * (items - cet)
    rows = lax.broadcasted_iota(jnp.int32, (L_ITEM * EMBED, L_ITEM), 0)
    cols = lax.broadcasted_iota(jnp.int32, (L_ITEM * EMBED, L_ITEM), 1)
    G = jnp.where(rows // EMBED == cols, 1.0, 0.0).astype(jnp.float32)
    d = jnp.dot(d2, G, preferred_element_type=jnp.float32)
    out_ref[...] = 1.0 / (1.0 + d)


def _tc_tail(summed, ctx_idx, st0, irows, W1, b1, W2, b2, W3, b3):
    BT = 256
    grid = (B // BT,)
    full = lambda shape: pl.BlockSpec(shape, lambda i: (0, 0))
    return pl.pallas_call(
        _tc_body,
        grid=grid,
        in_specs=[
            pl.BlockSpec((BT, EMBED), lambda i: (i, 0)),
            pl.BlockSpec((BT, L_CTX), lambda i: (i, 0)),
            full((1, EMBED)),
            pl.BlockSpec((BT, L_ITEM * EMBED), lambda i: (i, 0)),
            full((CTX, 2 * CTX)),
            full((1, 2 * CTX)),
            full((2 * CTX, 4 * CTX)),
            full((1, 4 * CTX)),
            full((4 * CTX, EMBED)),
            full((1, EMBED)),
        ],
        out_specs=pl.BlockSpec((BT, L_ITEM), lambda i: (i, 0)),
        out_shape=jax.ShapeDtypeStruct((B, L_ITEM), jnp.float32),
    )(summed, ctx_idx, st0, irows, W1, b1, W2, b2, W3, b3)


def kernel(item_indices, context_indices, item_table, set_table,
           W1, b1, W2, b2, W3, b3):
    ctx_i32 = context_indices.astype(jnp.int32)
    ctx_adj = jnp.maximum(ctx_i32 - 1, 0).reshape(-1)
    iidx = item_indices.astype(jnp.int32).reshape(-1)
    summed, irows = _sc_gather_pool(iidx, ctx_adj, item_table, set_table)
    out = _tc_tail(summed, ctx_i32, set_table[0:1],
                   irows.reshape(B, L_ITEM * EMBED),
                   W1, b1.reshape(1, -1), W2, b2.reshape(1, -1),
                   W3, b3.reshape(1, -1))
    return out
